# baseline (speedup 1.0000x reference)
import dataclasses
import functools
import jax
import jax.numpy as jnp
from jax import lax
from jax.experimental import pallas as pl
from jax.experimental.pallas import tpu as pltpu
from jax.experimental.pallas import tpu_sc as plsc

ALPHA = 0.15
KSTEPS = 2
SCALING = 1.8

def _sc_compiler_params():
    cp = pltpu.CompilerParams()
    if "needs_layout_passes" in pltpu.CompilerParams.__dataclass_fields__:
        cp = dataclasses.replace(cp, needs_layout_passes=False)
    return cp


NC = 2
NS = 16
NW = NC * NS
LANES = 16


def _linear_norm_body(x_ref, w_ref, b_ref, o_ref):
    h = lax.dot_general(
        x_ref[...], w_ref[...], (((1,), (1,)), ((), ())),
        preferred_element_type=jnp.float32,
    )
    h = h + b_ref[...]
    nrm = jnp.sqrt(jnp.sum(h * h, axis=1, keepdims=True))
    o_ref[...] = h * (SCALING / jnp.maximum(nrm, 1e-12))


def _linear_norm(x, w, b2):
    n, d = x.shape
    br = 2000
    return pl.pallas_call(
        _linear_norm_body,
        grid=(n // br,),
        in_specs=[
            pl.BlockSpec((br, d), lambda i: (i, 0)),
            pl.BlockSpec((d, d), lambda i: (0, 0)),
            pl.BlockSpec((1, d), lambda i: (0, 0)),
        ],
        out_specs=pl.BlockSpec((br, d), lambda i: (i, 0)),
        out_shape=jax.ShapeDtypeStruct((n, d), jnp.float32),
    )(x, w, b2)


def _make_hist_kernel(n, e, rows128):
    e_per_w = e // NW
    mesh = plsc.VectorSubcoreMesh(core_axis_name="c", subcore_axis_name="s")

    @functools.partial(
        pl.kernel,
        out_type=jax.ShapeDtypeStruct((NW, rows128, 128), jnp.float32),
        mesh=mesh,
        scratch_types=[
            pltpu.VMEM((rows128, 128), jnp.float32),
            pltpu.VMEM((e_per_w // 128, 128), jnp.int32),
        ],
        compiler_params=_sc_compiler_params(),
    )
    def hist_kernel(ei_hbm, zeros_hbm, out_hbm, hist, didx):
        c = lax.axis_index("c")
        s = lax.axis_index("s")
        wid = c * NS + s
        pltpu.sync_copy(zeros_hbm, hist)
        pltpu.sync_copy(ei_hbm.at[1, wid], didx)
        ones = jnp.full((LANES,), 1.0, jnp.float32)

        @pl.loop(0, e_per_w // 128)
        def _(i):
            for l in range(128 // LANES):
                idx = didx[i, pl.ds(l * LANES, LANES)]
                row = idx >> 7
                lane = idx & 127
                plsc.addupdate_scatter(hist, [row, lane], ones)

        pltpu.sync_copy(hist, out_hbm.at[wid])

    return hist_kernel


def _dis_body(histp_ref, dis_ref):
    deg = jnp.sum(histp_ref[...], axis=0) + 1.0
    dis_ref[...] = lax.rsqrt(deg)


def _dis_tc(histp):
    nw, rows128, w = histp.shape
    return pl.pallas_call(
        _dis_body,
        grid=(1,),
        in_specs=[pl.BlockSpec((nw, rows128, w), lambda i: (0, 0, 0))],
        out_specs=pl.BlockSpec((rows128, w), lambda i: (0, 0)),
        out_shape=jax.ShapeDtypeStruct((rows128, w), jnp.float32),
    )(histp)


def _prep_body(dis_ref, hs_ref, u_ref):
    u_ref[...] = hs_ref[...] * dis_ref[...]


def _prep(dis2, hs):
    n, d = hs.shape
    br = 2000
    return pl.pallas_call(
        _prep_body,
        grid=(n // br,),
        in_specs=[
            pl.BlockSpec((br, 1), lambda i: (i, 0)),
            pl.BlockSpec((br, d), lambda i: (i, 0)),
        ],
        out_specs=pl.BlockSpec((br, d), lambda i: (i, 0)),
        out_shape=jax.ShapeDtypeStruct((n, d), jnp.float32),
    )(dis2, hs)


_CHUNK = 128
_NBUF = 2
_SLAB = 8


def _make_propagate_kernel(n_pad, e_pad, d):
    e_per_w = e_pad // NW
    n_chunks = e_per_w // _CHUNK
    n_slabs = n_chunks // _SLAB
    assert n_slabs % 2 == 0 and n_chunks % _SLAB == 0
    rows_per_s = n_pad // NS
    mesh = plsc.VectorSubcoreMesh(core_axis_name="c", subcore_axis_name="s")

    @functools.partial(
        pl.kernel,
        out_type=jax.ShapeDtypeStruct((NC, n_pad, d), jnp.float32),
        mesh=mesh,
        scratch_types=[
            pltpu.VMEM_SHARED((n_pad, d), jnp.float32),
            [pltpu.VMEM((_CHUNK, d), jnp.float32) for _ in range(_NBUF)],
            [pltpu.VMEM((_SLAB, _CHUNK), jnp.int32) for _ in range(2)],
            [pltpu.VMEM((_SLAB, _CHUNK), jnp.int32) for _ in range(2)],
            [pltpu.SemaphoreType.DMA for _ in range(_NBUF)],
            [pltpu.SemaphoreType.DMA for _ in range(2)],
        ],
    )
    def prop_kernel(u_hbm, ei_hbm, zeros_hbm, out_hbm,
                    acc, bufs, sslab, dslab, gsems, ssems):
        c = lax.axis_index("c")
        s = lax.axis_index("s")
        wid = c * NS + s
        my_rows = pl.ds(s * rows_per_s, rows_per_s)
        src_w = ei_hbm.at[0, wid]
        dst_w = ei_hbm.at[1, wid]

        def fire_slab(sg, k):
            rows = pl.ds(sg * _SLAB, _SLAB)
            pltpu.async_copy(src_w.at[rows], sslab[k], ssems[k])
            pltpu.async_copy(dst_w.at[rows], dslab[k], ssems[k])

        def wait_slab(sg, k):
            rows = pl.ds(sg * _SLAB, _SLAB)
            pltpu.make_async_copy(src_w.at[rows], sslab[k], ssems[k]).wait()
            pltpu.make_async_copy(dst_w.at[rows], dslab[k], ssems[k]).wait()

        def fire_gather(k, j, b):
            pltpu.async_copy(u_hbm.at[sslab[k].at[j]], bufs[b], gsems[b])

        def wait_gather(k, j, b):
            pltpu.make_async_copy(
                u_hbm.at[sslab[k].at[j]], bufs[b], gsems[b]).wait()

        fire_slab(0, 0)
        fire_slab(1, 1)
        zb = zeros_hbm.shape[0]
        nz, rz = divmod(rows_per_s, zb)
        pltpu.sync_copy(zeros_hbm, bufs[0].at[pl.ds(0, zb)])
        for kk in range(nz):
            pltpu.sync_copy(bufs[0].at[pl.ds(0, zb)],
                            acc.at[pl.ds(s * rows_per_s + kk * zb, zb)])
        if rz:
            pltpu.sync_copy(bufs[0].at[pl.ds(0, rz)],
                            acc.at[pl.ds(s * rows_per_s + nz * zb, rz)])
        plsc.subcore_barrier()
        wait_slab(0, 0)
        fire_gather(0, 0, 0)
        fire_gather(0, 1, 1)

        @pl.loop(0, n_slabs // 2)
        def _(p):
            for kk in range(2):
                sg = p * 2 + kk
                for j in range(_SLAB):
                    cur = sg * _SLAB + j
                    b = j % _NBUF
                    wait_gather(kk, j, b)
                    pltpu.sync_copy(bufs[b], acc.at[dslab[kk].at[j]],
                                    add=True)
                    if j < _SLAB - _NBUF:
                        @pl.when(cur + _NBUF < n_chunks)
                        def _():
                            fire_gather(kk, j + _NBUF, b)
                    else:
                        if j == _SLAB - _NBUF:
                            @pl.when(sg + 1 < n_slabs)
                            def _():
                                wait_slab(sg + 1, 1 - kk)
                        @pl.when(cur + _NBUF < n_chunks)
                        def _():
                            fire_gather(1 - kk, j + _NBUF - _SLAB, b)

                @pl.when(sg + 2 < n_slabs)
                def _():
                    fire_slab(sg + 2, kk)

        plsc.subcore_barrier()
        pltpu.sync_copy(acc.at[my_rows], out_hbm.at[c].at[my_rows])

    return prop_kernel


def _combine_body(part_ref, prev_ref, hs_ref, dis_ref, out_ref, u_ref=None):
    agg = part_ref[0] + part_ref[1]
    dis = dis_ref[...]
    out = (1.0 - ALPHA) * (dis * agg + (dis * dis) * prev_ref[...]) \
        + ALPHA * hs_ref[...]
    out_ref[...] = out
    if u_ref is not None:
        u_ref[...] = out * dis


def _combine(part, prev, hs, dis, want_u):
    n, d = hs.shape
    br = 2000
    blk = pl.BlockSpec((br, d), lambda i: (i, 0))
    n_out = 2 if want_u else 1
    return pl.pallas_call(
        _combine_body,
        grid=(n // br,),
        in_specs=[
            pl.BlockSpec((NC, br, d), lambda i: (0, i, 0)),
            blk, blk,
            pl.BlockSpec((br, 1), lambda i: (i, 0)),
        ],
        out_specs=[blk] * n_out,
        out_shape=[jax.ShapeDtypeStruct((n, d), jnp.float32)] * n_out,
    )(part, prev, hs, dis)


def kernel(x, edge_index, W, b):
    n, d = x.shape
    e = edge_index.shape[1]
    assert n % LANES == 0 and n % NS == 0 and n % 1000 == 0
    assert e % NW == 0

    n_pad = ((n + NS * 8 - 1) // (NS * 8)) * (NS * 8)
    if n_pad == n:
        n_pad += NS * 8
    egrain = NW * _CHUNK * _SLAB * 2
    e_pad = (e + egrain - 1) // egrain * egrain
    npad_e = e_pad - e

    rows128 = (n + 127) // 128
    assert rows128 * 128 >= n and n_pad <= rows128 * 128

    b2 = b.reshape(1, d)
    zeros_nd = jnp.zeros((88, d), jnp.float32)
    zeros_hist = jnp.zeros((rows128, 128), jnp.float32)

    fill = jnp.arange(npad_e, dtype=jnp.int32)
    ei_pad = jnp.concatenate(
        [edge_index,
         jnp.stack([fill % n, n + fill % (n_pad - n)])], axis=1)
    ei4 = ei_pad.reshape(2, NW, e_pad // (NW * _CHUNK), _CHUNK)

    hs = _linear_norm(x, W, b2)
    histp = _make_hist_kernel(n, e_pad, rows128)(ei4, zeros_hist)
    dis = _dis_tc(histp).reshape(rows128 * 128, 1)
    u = _prep(dis, hs)

    out = hs
    prop = _make_propagate_kernel(n_pad, e_pad, d)
    for step in range(KSTEPS):
        part = prop(u, ei4, zeros_nd)
        res = _combine(part, out, hs, dis, want_u=step < KSTEPS - 1)
        out = res[0]
        u = res[1] if len(res) > 1 else None
    return out


# speedup vs baseline: 1.0892x; 1.0892x over previous
import dataclasses
import functools
import jax
import jax.numpy as jnp
from jax import lax
from jax.experimental import pallas as pl
from jax.experimental.pallas import tpu as pltpu
from jax.experimental.pallas import tpu_sc as plsc

ALPHA = 0.15
KSTEPS = 2
SCALING = 1.8

def _sc_compiler_params():
    cp = pltpu.CompilerParams()
    if "needs_layout_passes" in pltpu.CompilerParams.__dataclass_fields__:
        cp = dataclasses.replace(cp, needs_layout_passes=False)
    return cp


NC = 2
NS = 16
NW = NC * NS
LANES = 16


def _linear_norm_body(x_ref, w_ref, b_ref, o_ref):
    h = lax.dot_general(
        x_ref[...], w_ref[...], (((1,), (1,)), ((), ())),
        preferred_element_type=jnp.float32,
    )
    h = h + b_ref[...]
    nrm = jnp.sqrt(jnp.sum(h * h, axis=1, keepdims=True))
    o_ref[...] = h * (SCALING / jnp.maximum(nrm, 1e-12))


def _linear_norm(x, w, b2):
    n, d = x.shape
    br = 2000
    return pl.pallas_call(
        _linear_norm_body,
        grid=(n // br,),
        in_specs=[
            pl.BlockSpec((br, d), lambda i: (i, 0)),
            pl.BlockSpec((d, d), lambda i: (0, 0)),
            pl.BlockSpec((1, d), lambda i: (0, 0)),
        ],
        out_specs=pl.BlockSpec((br, d), lambda i: (i, 0)),
        out_shape=jax.ShapeDtypeStruct((n, d), jnp.float32),
    )(x, w, b2)


def _make_hist_kernel(n, e, rows128):
    e_per_w = e // NW
    mesh = plsc.VectorSubcoreMesh(core_axis_name="c", subcore_axis_name="s")

    @functools.partial(
        pl.kernel,
        out_type=jax.ShapeDtypeStruct((NW, rows128, 128), jnp.float32),
        mesh=mesh,
        scratch_types=[
            pltpu.VMEM((rows128, 128), jnp.float32),
            pltpu.VMEM((e_per_w // _CHUNK, _CHUNK), jnp.int32),
        ],
        compiler_params=_sc_compiler_params(),
    )
    def hist_kernel(ei_hbm, zeros_hbm, out_hbm, hist, didx):
        c = lax.axis_index("c")
        s = lax.axis_index("s")
        wid = c * NS + s
        pltpu.sync_copy(zeros_hbm, hist)
        pltpu.sync_copy(ei_hbm.at[1, wid], didx)
        ones = jnp.full((LANES,), 1.0, jnp.float32)

        @pl.loop(0, e_per_w // _CHUNK)
        def _(i):
            for l in range(_CHUNK // LANES):
                idx = didx[i, pl.ds(l * LANES, LANES)]
                row = idx >> 7
                lane = idx & 127
                plsc.addupdate_scatter(hist, [row, lane], ones)

        pltpu.sync_copy(hist, out_hbm.at[wid])

    return hist_kernel


def _dis_body(histp_ref, dis_ref):
    deg = jnp.sum(histp_ref[...], axis=0) + 1.0
    dis_ref[...] = lax.rsqrt(deg)


def _dis_tc(histp):
    nw, rows128, w = histp.shape
    return pl.pallas_call(
        _dis_body,
        grid=(1,),
        in_specs=[pl.BlockSpec((nw, rows128, w), lambda i: (0, 0, 0))],
        out_specs=pl.BlockSpec((rows128, w), lambda i: (0, 0)),
        out_shape=jax.ShapeDtypeStruct((rows128, w), jnp.float32),
    )(histp)


def _prep_body(dis_ref, hs_ref, u_ref):
    u_ref[...] = hs_ref[...] * dis_ref[...]


def _prep(dis2, hs):
    n, d = hs.shape
    br = 2000
    return pl.pallas_call(
        _prep_body,
        grid=(n // br,),
        in_specs=[
            pl.BlockSpec((br, 1), lambda i: (i, 0)),
            pl.BlockSpec((br, d), lambda i: (i, 0)),
        ],
        out_specs=pl.BlockSpec((br, d), lambda i: (i, 0)),
        out_shape=jax.ShapeDtypeStruct((n, d), jnp.float32),
    )(dis2, hs)


_CHUNK = 80
_NBUF = 4
_SLAB = 8


def _make_propagate_kernel(n_pad, e_pad, d):
    e_per_w = e_pad // NW
    n_chunks = e_per_w // _CHUNK
    n_slabs = n_chunks // _SLAB
    assert n_slabs % 2 == 0 and n_chunks % _SLAB == 0
    rows_per_s = n_pad // NS
    mesh = plsc.VectorSubcoreMesh(core_axis_name="c", subcore_axis_name="s")

    @functools.partial(
        pl.kernel,
        out_type=jax.ShapeDtypeStruct((NC, n_pad, d), jnp.float32),
        mesh=mesh,
        scratch_types=[
            pltpu.VMEM_SHARED((n_pad, d), jnp.float32),
            [pltpu.VMEM((_CHUNK, d), jnp.float32) for _ in range(_NBUF)],
            [pltpu.VMEM((_SLAB, _CHUNK), jnp.int32) for _ in range(2)],
            [pltpu.VMEM((_SLAB, _CHUNK), jnp.int32) for _ in range(2)],
            [pltpu.SemaphoreType.DMA for _ in range(_NBUF)],
            [pltpu.SemaphoreType.DMA for _ in range(2)],
        ],
    )
    def prop_kernel(u_hbm, ei_hbm, zeros_hbm, out_hbm,
                    acc, bufs, sslab, dslab, gsems, ssems):
        c = lax.axis_index("c")
        s = lax.axis_index("s")
        wid = c * NS + s
        my_rows = pl.ds(s * rows_per_s, rows_per_s)
        src_w = ei_hbm.at[0, wid]
        dst_w = ei_hbm.at[1, wid]

        def fire_slab(sg, k):
            rows = pl.ds(sg * _SLAB, _SLAB)
            pltpu.async_copy(src_w.at[rows], sslab[k], ssems[k])
            pltpu.async_copy(dst_w.at[rows], dslab[k], ssems[k])

        def wait_slab(sg, k):
            rows = pl.ds(sg * _SLAB, _SLAB)
            pltpu.make_async_copy(src_w.at[rows], sslab[k], ssems[k]).wait()
            pltpu.make_async_copy(dst_w.at[rows], dslab[k], ssems[k]).wait()

        def fire_gather(k, j, b):
            pltpu.async_copy(u_hbm.at[sslab[k].at[j]], bufs[b], gsems[b])

        def wait_gather(k, j, b):
            pltpu.make_async_copy(
                u_hbm.at[sslab[k].at[j]], bufs[b], gsems[b]).wait()

        fire_slab(0, 0)
        fire_slab(1, 1)
        zb = zeros_hbm.shape[0]
        nz, rz = divmod(rows_per_s, zb)
        pltpu.sync_copy(zeros_hbm, bufs[0].at[pl.ds(0, zb)])
        for kk in range(nz):
            pltpu.sync_copy(bufs[0].at[pl.ds(0, zb)],
                            acc.at[pl.ds(s * rows_per_s + kk * zb, zb)])
        if rz:
            pltpu.sync_copy(bufs[0].at[pl.ds(0, rz)],
                            acc.at[pl.ds(s * rows_per_s + nz * zb, rz)])
        plsc.subcore_barrier()
        wait_slab(0, 0)
        for j in range(_NBUF):
            fire_gather(0, j, j)

        @pl.loop(0, n_slabs // 2)
        def _(p):
            for kk in range(2):
                sg = p * 2 + kk
                for j in range(_SLAB):
                    cur = sg * _SLAB + j
                    b = j % _NBUF
                    wait_gather(kk, j, b)
                    pltpu.sync_copy(bufs[b], acc.at[dslab[kk].at[j]],
                                    add=True)
                    if j < _SLAB - _NBUF:
                        @pl.when(cur + _NBUF < n_chunks)
                        def _():
                            fire_gather(kk, j + _NBUF, b)
                    else:
                        if j == _SLAB - _NBUF:
                            @pl.when(sg + 1 < n_slabs)
                            def _():
                                wait_slab(sg + 1, 1 - kk)
                        @pl.when(cur + _NBUF < n_chunks)
                        def _():
                            fire_gather(1 - kk, j + _NBUF - _SLAB, b)

                @pl.when(sg + 2 < n_slabs)
                def _():
                    fire_slab(sg + 2, kk)

        plsc.subcore_barrier()
        pltpu.sync_copy(acc.at[my_rows], out_hbm.at[c].at[my_rows])

    return prop_kernel


def _combine_body(part_ref, prev_ref, hs_ref, dis_ref, out_ref, u_ref=None):
    agg = part_ref[0] + part_ref[1]
    dis = dis_ref[...]
    out = (1.0 - ALPHA) * (dis * agg + (dis * dis) * prev_ref[...]) \
        + ALPHA * hs_ref[...]
    out_ref[...] = out
    if u_ref is not None:
        u_ref[...] = out * dis


def _combine(part, prev, hs, dis, want_u):
    n, d = hs.shape
    br = 2000
    blk = pl.BlockSpec((br, d), lambda i: (i, 0))
    n_out = 2 if want_u else 1
    return pl.pallas_call(
        _combine_body,
        grid=(n // br,),
        in_specs=[
            pl.BlockSpec((NC, br, d), lambda i: (0, i, 0)),
            blk, blk,
            pl.BlockSpec((br, 1), lambda i: (i, 0)),
        ],
        out_specs=[blk] * n_out,
        out_shape=[jax.ShapeDtypeStruct((n, d), jnp.float32)] * n_out,
    )(part, prev, hs, dis)


def kernel(x, edge_index, W, b):
    n, d = x.shape
    e = edge_index.shape[1]
    assert n % LANES == 0 and n % NS == 0 and n % 1000 == 0
    assert e % NW == 0

    n_pad = ((n + NS * 8 - 1) // (NS * 8)) * (NS * 8)
    if n_pad == n:
        n_pad += NS * 8
    egrain = NW * _CHUNK * _SLAB * 2
    e_pad = (e + egrain - 1) // egrain * egrain
    npad_e = e_pad - e

    rows128 = (n + 127) // 128
    assert rows128 * 128 >= n and n_pad <= rows128 * 128

    b2 = b.reshape(1, d)
    zeros_nd = jnp.zeros((_CHUNK, d), jnp.float32)
    zeros_hist = jnp.zeros((rows128, 128), jnp.float32)

    fill = jnp.arange(npad_e, dtype=jnp.int32)
    ei_pad = jnp.concatenate(
        [edge_index,
         jnp.stack([fill % n, n + fill % (n_pad - n)])], axis=1)
    ei4 = ei_pad.reshape(2, NW, e_pad // (NW * _CHUNK), _CHUNK)

    hs = _linear_norm(x, W, b2)
    histp = _make_hist_kernel(n, e_pad, rows128)(ei4, zeros_hist)
    dis = _dis_tc(histp).reshape(rows128 * 128, 1)
    u = _prep(dis, hs)

    out = hs
    prop = _make_propagate_kernel(n_pad, e_pad, d)
    for step in range(KSTEPS):
        part = prop(u, ei4, zeros_nd)
        res = _combine(part, out, hs, dis, want_u=step < KSTEPS - 1)
        out = res[0]
        u = res[1] if len(res) > 1 else None
    return out


# speedup vs baseline: 1.1118x; 1.0208x over previous
import dataclasses
import functools
import jax
import jax.numpy as jnp
from jax import lax
from jax.experimental import pallas as pl
from jax.experimental.pallas import tpu as pltpu
from jax.experimental.pallas import tpu_sc as plsc

ALPHA = 0.15
KSTEPS = 2
SCALING = 1.8

def _sc_compiler_params():
    cp = pltpu.CompilerParams()
    if "needs_layout_passes" in pltpu.CompilerParams.__dataclass_fields__:
        cp = dataclasses.replace(cp, needs_layout_passes=False)
    return cp


NC = 2
NS = 16
NW = NC * NS
LANES = 16


def _linear_norm_body(x_ref, w_ref, b_ref, o_ref):
    h = lax.dot_general(
        x_ref[...], w_ref[...], (((1,), (1,)), ((), ())),
        preferred_element_type=jnp.float32,
    )
    h = h + b_ref[...]
    nrm = jnp.sqrt(jnp.sum(h * h, axis=1, keepdims=True))
    o_ref[...] = h * (SCALING / jnp.maximum(nrm, 1e-12))


def _linear_norm(x, w, b2):
    n, d = x.shape
    br = 5000
    return pl.pallas_call(
        _linear_norm_body,
        grid=(n // br,),
        in_specs=[
            pl.BlockSpec((br, d), lambda i: (i, 0)),
            pl.BlockSpec((d, d), lambda i: (0, 0)),
            pl.BlockSpec((1, d), lambda i: (0, 0)),
        ],
        out_specs=pl.BlockSpec((br, d), lambda i: (i, 0)),
        out_shape=jax.ShapeDtypeStruct((n, d), jnp.float32),
    )(x, w, b2)


def _make_hist_kernel(n, e, rows128):
    e_per_w = e // NW
    mesh = plsc.VectorSubcoreMesh(core_axis_name="c", subcore_axis_name="s")

    @functools.partial(
        pl.kernel,
        out_type=jax.ShapeDtypeStruct((NW, rows128, 128), jnp.float32),
        mesh=mesh,
        scratch_types=[
            pltpu.VMEM((rows128, 128), jnp.float32),
            pltpu.VMEM((e_per_w // _CHUNK, _CHUNK), jnp.int32),
        ],
        compiler_params=_sc_compiler_params(),
    )
    def hist_kernel(ei_hbm, zeros_hbm, out_hbm, hist, didx):
        c = lax.axis_index("c")
        s = lax.axis_index("s")
        wid = c * NS + s
        pltpu.sync_copy(zeros_hbm, hist)
        pltpu.sync_copy(ei_hbm.at[1, wid], didx)
        ones = jnp.full((LANES,), 1.0, jnp.float32)

        @pl.loop(0, e_per_w // _CHUNK)
        def _(i):
            for l in range(_CHUNK // LANES):
                idx = didx[i, pl.ds(l * LANES, LANES)]
                row = idx >> 7
                lane = idx & 127
                plsc.addupdate_scatter(hist, [row, lane], ones)

        pltpu.sync_copy(hist, out_hbm.at[wid])

    return hist_kernel


def _dis_body(histp_ref, dis_ref):
    deg = jnp.sum(histp_ref[...], axis=0) + 1.0
    dis_ref[...] = lax.rsqrt(deg)


def _dis_tc(histp):
    nw, rows128, w = histp.shape
    return pl.pallas_call(
        _dis_body,
        grid=(1,),
        in_specs=[pl.BlockSpec((nw, rows128, w), lambda i: (0, 0, 0))],
        out_specs=pl.BlockSpec((rows128, w), lambda i: (0, 0)),
        out_shape=jax.ShapeDtypeStruct((rows128, w), jnp.float32),
    )(histp)


def _prep_body(dis_ref, hs_ref, u_ref):
    u_ref[...] = hs_ref[...] * dis_ref[...]


def _prep(dis2, hs):
    n, d = hs.shape
    br = 5000
    return pl.pallas_call(
        _prep_body,
        grid=(n // br,),
        in_specs=[
            pl.BlockSpec((br, 1), lambda i: (i, 0)),
            pl.BlockSpec((br, d), lambda i: (i, 0)),
        ],
        out_specs=pl.BlockSpec((br, d), lambda i: (i, 0)),
        out_shape=jax.ShapeDtypeStruct((n, d), jnp.float32),
    )(dis2, hs)


_CHUNK = 80
_NBUF = 4
_SLAB = 8


def _make_propagate_kernel(n_pad, e_pad, d):
    e_per_w = e_pad // NW
    n_chunks = e_per_w // _CHUNK
    n_slabs = n_chunks // _SLAB
    assert n_slabs % 2 == 0 and n_chunks % _SLAB == 0
    rows_per_s = n_pad // NS
    mesh = plsc.VectorSubcoreMesh(core_axis_name="c", subcore_axis_name="s")

    @functools.partial(
        pl.kernel,
        out_type=jax.ShapeDtypeStruct((NC, n_pad, d), jnp.float32),
        mesh=mesh,
        scratch_types=[
            pltpu.VMEM_SHARED((n_pad, d), jnp.float32),
            [pltpu.VMEM((_CHUNK, d), jnp.float32) for _ in range(_NBUF)],
            [pltpu.VMEM((_SLAB, _CHUNK), jnp.int32) for _ in range(2)],
            [pltpu.VMEM((_SLAB, _CHUNK), jnp.int32) for _ in range(2)],
            [pltpu.SemaphoreType.DMA for _ in range(_NBUF)],
            [pltpu.SemaphoreType.DMA for _ in range(2)],
        ],
    )
    def prop_kernel(u_hbm, ei_hbm, zeros_hbm, out_hbm,
                    acc, bufs, sslab, dslab, gsems, ssems):
        c = lax.axis_index("c")
        s = lax.axis_index("s")
        wid = c * NS + s
        my_rows = pl.ds(s * rows_per_s, rows_per_s)
        src_w = ei_hbm.at[0, wid]
        dst_w = ei_hbm.at[1, wid]

        def fire_slab(sg, k):
            rows = pl.ds(sg * _SLAB, _SLAB)
            pltpu.async_copy(src_w.at[rows], sslab[k], ssems[k])
            pltpu.async_copy(dst_w.at[rows], dslab[k], ssems[k])

        def wait_slab(sg, k):
            rows = pl.ds(sg * _SLAB, _SLAB)
            pltpu.make_async_copy(src_w.at[rows], sslab[k], ssems[k]).wait()
            pltpu.make_async_copy(dst_w.at[rows], dslab[k], ssems[k]).wait()

        def fire_gather(k, j, b):
            pltpu.async_copy(u_hbm.at[sslab[k].at[j]], bufs[b], gsems[b])

        def wait_gather(k, j, b):
            pltpu.make_async_copy(
                u_hbm.at[sslab[k].at[j]], bufs[b], gsems[b]).wait()

        fire_slab(0, 0)
        fire_slab(1, 1)
        zb = zeros_hbm.shape[0]
        nz, rz = divmod(rows_per_s, zb)
        pltpu.sync_copy(zeros_hbm, bufs[0].at[pl.ds(0, zb)])
        for kk in range(nz):
            pltpu.sync_copy(bufs[0].at[pl.ds(0, zb)],
                            acc.at[pl.ds(s * rows_per_s + kk * zb, zb)])
        if rz:
            pltpu.sync_copy(bufs[0].at[pl.ds(0, rz)],
                            acc.at[pl.ds(s * rows_per_s + nz * zb, rz)])
        plsc.subcore_barrier()
        wait_slab(0, 0)
        for j in range(_NBUF):
            fire_gather(0, j, j)

        @pl.loop(0, n_slabs // 2)
        def _(p):
            for kk in range(2):
                sg = p * 2 + kk
                for j in range(_SLAB):
                    cur = sg * _SLAB + j
                    b = j % _NBUF
                    wait_gather(kk, j, b)
                    pltpu.sync_copy(bufs[b], acc.at[dslab[kk].at[j]],
                                    add=True)
                    if j < _SLAB - _NBUF:
                        @pl.when(cur + _NBUF < n_chunks)
                        def _():
                            fire_gather(kk, j + _NBUF, b)
                    else:
                        if j == _SLAB - _NBUF:
                            @pl.when(sg + 1 < n_slabs)
                            def _():
                                wait_slab(sg + 1, 1 - kk)
                        @pl.when(cur + _NBUF < n_chunks)
                        def _():
                            fire_gather(1 - kk, j + _NBUF - _SLAB, b)

                @pl.when(sg + 2 < n_slabs)
                def _():
                    fire_slab(sg + 2, kk)

        plsc.subcore_barrier()
        pltpu.sync_copy(acc.at[my_rows], out_hbm.at[c].at[my_rows])

    return prop_kernel


def _combine_v_body(part_ref, v_ref, w_ref, dis_ref, o_ref):
    agg = part_ref[0] + part_ref[1] + v_ref[...]
    dis = dis_ref[...]
    o_ref[...] = (1.0 - ALPHA) * (dis * dis) * agg + ALPHA * w_ref[...]


def _combine_f_body(part_ref, v_ref, hs_ref, dis_ref, o_ref):
    agg = part_ref[0] + part_ref[1] + v_ref[...]
    o_ref[...] = (1.0 - ALPHA) * dis_ref[...] * agg + ALPHA * hs_ref[...]


def _combine(part, v, dense, dis, final):
    n, d = dense.shape
    br = 5000
    blk = pl.BlockSpec((br, d), lambda i: (i, 0))
    return pl.pallas_call(
        _combine_f_body if final else _combine_v_body,
        grid=(n // br,),
        in_specs=[
            pl.BlockSpec((NC, br, d), lambda i: (0, i, 0)),
            blk, blk,
            pl.BlockSpec((br, 1), lambda i: (i, 0)),
        ],
        out_specs=blk,
        out_shape=jax.ShapeDtypeStruct((n, d), jnp.float32),
    )(part, v, dense, dis)


def kernel(x, edge_index, W, b):
    n, d = x.shape
    e = edge_index.shape[1]
    assert n % LANES == 0 and n % NS == 0 and n % 1000 == 0
    assert e % NW == 0

    n_pad = ((n + NS * 8 - 1) // (NS * 8)) * (NS * 8)
    if n_pad == n:
        n_pad += NS * 8
    egrain = NW * _CHUNK * _SLAB * 2
    e_pad = (e + egrain - 1) // egrain * egrain
    npad_e = e_pad - e

    rows128 = (n + 127) // 128
    assert rows128 * 128 >= n and n_pad <= rows128 * 128

    b2 = b.reshape(1, d)
    zeros_nd = jnp.zeros((_CHUNK, d), jnp.float32)
    zeros_hist = jnp.zeros((rows128, 128), jnp.float32)

    fill = jnp.arange(npad_e, dtype=jnp.int32)
    ei_pad = jnp.concatenate(
        [edge_index,
         jnp.stack([fill % n, n + fill % (n_pad - n)])], axis=1)
    ei4 = ei_pad.reshape(2, NW, e_pad // (NW * _CHUNK), _CHUNK)

    hs = _linear_norm(x, W, b2)
    histp = _make_hist_kernel(n, e_pad, rows128)(ei4, zeros_hist)
    dis = _dis_tc(histp).reshape(rows128 * 128, 1)
    w = _prep(dis, hs)

    v = w
    prop = _make_propagate_kernel(n_pad, e_pad, d)
    for step in range(KSTEPS):
        part = prop(v, ei4, zeros_nd)
        final = step == KSTEPS - 1
        v = _combine(part, v, hs if final else w, dis, final)
    return v
